# baseline (speedup 1.0000x reference)
import functools

import jax
import jax.numpy as jnp
from jax import lax
from jax.experimental import pallas as pl
from jax.experimental.pallas import tpu as pltpu
from jax.experimental.pallas import tpu_sc as plsc

STATE_DIM = 1024
BUFFER_SIZE = 8192
N_OUT = 2048
STATE_ROWS = 16384

_NC = 2
_NS = 16
_NW = _NC * _NS
_ROWS_PER_W = N_OUT // _NW

_SC_ROWS = 4096
_SROWS_PER_W = _SC_ROWS // _NW
_ACC_ROWS = 16
_NPASS = _SROWS_PER_W // _ACC_ROWS

_CHUNK = 2048
_TC_ROWS = STATE_ROWS - _SC_ROWS
_GRID = _TC_ROWS // _CHUNK
_BLK_OFF = _SC_ROWS // _CHUNK


def _tc_sum_body(x_ref, o_ref):
    i = pl.program_id(0)
    part = jnp.sum(x_ref[...], axis=0, keepdims=True)

    @pl.when(i == 0)
    def _():
        o_ref[...] = part

    @pl.when(i > 0)
    def _():
        o_ref[...] += part


_tc_sum_call = pl.pallas_call(
    _tc_sum_body,
    grid=(_GRID,),
    in_specs=[pl.BlockSpec((_CHUNK, STATE_DIM), lambda i: (i + _BLK_OFF, 0))],
    out_specs=pl.BlockSpec((1, STATE_DIM), lambda i: (0, 0)),
    out_shape=jax.ShapeDtypeStruct((1, STATE_DIM), jnp.float32),
)


_sc_mesh = plsc.VectorSubcoreMesh(
    core_axis_name="c", subcore_axis_name="s", num_cores=_NC, num_subcores=_NS
)


@functools.partial(
    pl.kernel,
    out_type=(
        jax.ShapeDtypeStruct((N_OUT, STATE_DIM), jnp.float32),
        jax.ShapeDtypeStruct((_NW, STATE_DIM), jnp.float32),
    ),
    mesh=_sc_mesh,
    scratch_types=[
        pltpu.VMEM((_ROWS_PER_W,), jnp.int32),
        pltpu.VMEM((_ROWS_PER_W, STATE_DIM), jnp.float32),
        pltpu.VMEM((_ACC_ROWS, STATE_DIM), jnp.float32),
        pltpu.VMEM((_ACC_ROWS, STATE_DIM), jnp.float32),
        pltpu.VMEM((1, STATE_DIM), jnp.float32),
        pltpu.SemaphoreType.DMA,
        pltpu.SemaphoreType.DMA,
        pltpu.SemaphoreType.DMA,
    ],
)
def _sc_gather_psum(state_hbm, table_hbm, idx_hbm, out_hbm, part_hbm,
                    idx_v, rows_v, buf0, buf1, psum_v, gsem, sem0, sem1):
    wid = lax.axis_index("s") * _NC + lax.axis_index("c")
    base = wid * _ROWS_PER_W
    sbase = wid * _SROWS_PER_W

    pltpu.sync_copy(idx_hbm.at[pl.ds(base, _ROWS_PER_W)], idx_v)
    gcopy = pltpu.async_copy(table_hbm.at[idx_v], rows_v, gsem)

    bufs = (buf0, buf1)
    sems = (sem0, sem1)

    def _acc_pass(buf, first):
        def body(k, carry):
            sl = pl.ds(k * 16, 16)
            s = buf[0, sl]
            for r in range(1, _ACC_ROWS):
                s = s + buf[r, sl]
            if not first:
                s = s + psum_v[0, sl]
            psum_v[0, sl] = s
            return carry

        lax.fori_loop(0, STATE_DIM // 16, body, 0)

    copies = [None, None]
    copies[0] = pltpu.async_copy(
        state_hbm.at[pl.ds(sbase, _ACC_ROWS)], buf0, sem0
    )
    for p in range(_NPASS):
        cur = p % 2
        if p + 1 < _NPASS:
            copies[1 - cur] = pltpu.async_copy(
                state_hbm.at[pl.ds(sbase + (p + 1) * _ACC_ROWS, _ACC_ROWS)],
                bufs[1 - cur], sems[1 - cur],
            )
        copies[cur].wait()
        _acc_pass(bufs[cur], first=(p == 0))

    pltpu.sync_copy(psum_v, part_hbm.at[pl.ds(wid, 1)])

    gcopy.wait()
    pltpu.sync_copy(rows_v, out_hbm.at[pl.ds(base, _ROWS_PER_W)])


def _patch_body(tc_ref, part_ref, g_ref, o_ref):
    total = tc_ref[...] + jnp.sum(part_ref[...], axis=0, keepdims=True)
    mean = total * jnp.float32(1.0 / STATE_ROWS)
    rows = lax.broadcasted_iota(jnp.int32, (8, STATE_DIM), 0)
    o_ref[...] = jnp.where(rows == 0, mean, g_ref[...])


_patch_call = pl.pallas_call(
    _patch_body,
    grid=(1,),
    in_specs=[
        pl.BlockSpec((1, STATE_DIM), lambda i: (0, 0)),
        pl.BlockSpec((_NW, STATE_DIM), lambda i: (0, 0)),
        pl.BlockSpec((8, STATE_DIM), lambda i: (0, 0)),
    ],
    out_specs=pl.BlockSpec((8, STATE_DIM), lambda i: (0, 0)),
    out_shape=jax.ShapeDtypeStruct((N_OUT, STATE_DIM), jnp.float32),
    input_output_aliases={2: 0},
)


def kernel(state, state_buffer, n, ptr):
    n = jnp.asarray(n, jnp.int32)
    ptr = jnp.asarray(ptr, jnp.int32)
    idx = (ptr - (n - N_OUT) - jnp.arange(N_OUT, dtype=jnp.int32)) % BUFFER_SIZE
    gathered, part = _sc_gather_psum(state, state_buffer, idx)
    tc_sum = _tc_sum_call(state)
    return _patch_call(tc_sum, part, gathered)


# speedup vs baseline: 1.0644x; 1.0644x over previous
import functools

import jax
import jax.numpy as jnp
from jax import lax
from jax.experimental import pallas as pl
from jax.experimental.pallas import tpu as pltpu
from jax.experimental.pallas import tpu_sc as plsc

STATE_DIM = 1024
BUFFER_SIZE = 8192
N_OUT = 2048
STATE_ROWS = 16384

_NC = 2
_NS = 16
_NW = _NC * _NS
_ROWS_PER_W = N_OUT // _NW

_SC_ROWS = 2048
_SROWS_PER_W = _SC_ROWS // _NW
_ACC_ROWS = 16
_NPASS = _SROWS_PER_W // _ACC_ROWS

_CHUNK = 2048
_TC_ROWS = STATE_ROWS - _SC_ROWS
_GRID = _TC_ROWS // _CHUNK
_BLK_OFF = _SC_ROWS // _CHUNK


def _tc_sum_body(x_ref, o_ref):
    i = pl.program_id(0)
    part = jnp.sum(x_ref[...], axis=0, keepdims=True)

    @pl.when(i == 0)
    def _():
        o_ref[...] = part

    @pl.when(i > 0)
    def _():
        o_ref[...] += part


_tc_sum_call = pl.pallas_call(
    _tc_sum_body,
    grid=(_GRID,),
    in_specs=[pl.BlockSpec((_CHUNK, STATE_DIM), lambda i: (i + _BLK_OFF, 0))],
    out_specs=pl.BlockSpec((1, STATE_DIM), lambda i: (0, 0)),
    out_shape=jax.ShapeDtypeStruct((1, STATE_DIM), jnp.float32),
)


_sc_mesh = plsc.VectorSubcoreMesh(
    core_axis_name="c", subcore_axis_name="s", num_cores=_NC, num_subcores=_NS
)


@functools.partial(
    pl.kernel,
    out_type=(
        jax.ShapeDtypeStruct((N_OUT, STATE_DIM), jnp.float32),
        jax.ShapeDtypeStruct((_NW, STATE_DIM), jnp.float32),
    ),
    mesh=_sc_mesh,
    scratch_types=[
        pltpu.VMEM((_ROWS_PER_W,), jnp.int32),
        pltpu.VMEM((_ROWS_PER_W, STATE_DIM), jnp.float32),
        pltpu.VMEM((_ACC_ROWS, STATE_DIM), jnp.float32),
        pltpu.VMEM((_ACC_ROWS, STATE_DIM), jnp.float32),
        pltpu.VMEM((1, STATE_DIM), jnp.float32),
        pltpu.SemaphoreType.DMA,
        pltpu.SemaphoreType.DMA,
        pltpu.SemaphoreType.DMA,
    ],
)
def _sc_gather_psum(state_hbm, table_hbm, idx_hbm, out_hbm, part_hbm,
                    idx_v, rows_v, buf0, buf1, psum_v, gsem, sem0, sem1):
    wid = lax.axis_index("s") * _NC + lax.axis_index("c")
    base = wid * _ROWS_PER_W
    sbase = wid * _SROWS_PER_W

    pltpu.sync_copy(idx_hbm.at[pl.ds(base, _ROWS_PER_W)], idx_v)
    gcopy = pltpu.async_copy(table_hbm.at[idx_v], rows_v, gsem)

    bufs = (buf0, buf1)
    sems = (sem0, sem1)

    def _acc_pass(buf, first):
        def body(k, carry):
            sl = pl.ds(k * 16, 16)
            s = buf[0, sl]
            for r in range(1, _ACC_ROWS):
                s = s + buf[r, sl]
            if not first:
                s = s + psum_v[0, sl]
            psum_v[0, sl] = s
            return carry

        lax.fori_loop(0, STATE_DIM // 16, body, 0)

    copies = [None, None]
    copies[0] = pltpu.async_copy(
        state_hbm.at[pl.ds(sbase, _ACC_ROWS)], buf0, sem0
    )
    for p in range(_NPASS):
        cur = p % 2
        if p + 1 < _NPASS:
            copies[1 - cur] = pltpu.async_copy(
                state_hbm.at[pl.ds(sbase + (p + 1) * _ACC_ROWS, _ACC_ROWS)],
                bufs[1 - cur], sems[1 - cur],
            )
        copies[cur].wait()
        _acc_pass(bufs[cur], first=(p == 0))

    pltpu.sync_copy(psum_v, part_hbm.at[pl.ds(wid, 1)])

    gcopy.wait()
    pltpu.sync_copy(rows_v, out_hbm.at[pl.ds(base, _ROWS_PER_W)])


def _patch_body(tc_ref, part_ref, g_ref, o_ref):
    total = tc_ref[...] + jnp.sum(part_ref[...], axis=0, keepdims=True)
    mean = total * jnp.float32(1.0 / STATE_ROWS)
    rows = lax.broadcasted_iota(jnp.int32, (8, STATE_DIM), 0)
    o_ref[...] = jnp.where(rows == 0, mean, g_ref[...])


_patch_call = pl.pallas_call(
    _patch_body,
    grid=(1,),
    in_specs=[
        pl.BlockSpec((1, STATE_DIM), lambda i: (0, 0)),
        pl.BlockSpec((_NW, STATE_DIM), lambda i: (0, 0)),
        pl.BlockSpec((8, STATE_DIM), lambda i: (0, 0)),
    ],
    out_specs=pl.BlockSpec((8, STATE_DIM), lambda i: (0, 0)),
    out_shape=jax.ShapeDtypeStruct((N_OUT, STATE_DIM), jnp.float32),
    input_output_aliases={2: 0},
)


def kernel(state, state_buffer, n, ptr):
    n = jnp.asarray(n, jnp.int32)
    ptr = jnp.asarray(ptr, jnp.int32)
    idx = (ptr - (n - N_OUT) - jnp.arange(N_OUT, dtype=jnp.int32)) % BUFFER_SIZE
    gathered, part = _sc_gather_psum(state, state_buffer, idx)
    tc_sum = _tc_sum_call(state)
    return _patch_call(tc_sum, part, gathered)


# speedup vs baseline: 1.0680x; 1.0034x over previous
import functools

import jax
import jax.numpy as jnp
from jax import lax
from jax.experimental import pallas as pl
from jax.experimental.pallas import tpu as pltpu
from jax.experimental.pallas import tpu_sc as plsc

STATE_DIM = 1024
BUFFER_SIZE = 8192
N_OUT = 2048
STATE_ROWS = 16384

_CHUNK = 4096
_GRID = STATE_ROWS // _CHUNK

_NC = 2
_NS = 16
_NW = _NC * _NS
_ROWS_PER_W = N_OUT // _NW


def _mean_body(x_ref, o_ref):
    i = pl.program_id(0)
    part = jnp.sum(x_ref[...], axis=0, keepdims=True)

    @pl.when(i == 0)
    def _():
        o_ref[...] = part

    @pl.when(i > 0)
    def _():
        o_ref[...] += part

    @pl.when(i == _GRID - 1)
    def _():
        o_ref[...] *= jnp.float32(1.0 / STATE_ROWS)


_mean_call = pl.pallas_call(
    _mean_body,
    grid=(_GRID,),
    in_specs=[pl.BlockSpec((_CHUNK, STATE_DIM), lambda i: (i, 0))],
    out_specs=pl.BlockSpec((1, STATE_DIM), lambda i: (0, 0)),
    out_shape=jax.ShapeDtypeStruct((1, STATE_DIM), jnp.float32),
)


_sc_mesh = plsc.VectorSubcoreMesh(
    core_axis_name="c", subcore_axis_name="s", num_cores=_NC, num_subcores=_NS
)


@functools.partial(
    pl.kernel,
    out_type=jax.ShapeDtypeStruct((N_OUT, STATE_DIM), jnp.float32),
    mesh=_sc_mesh,
    scratch_types=[
        pltpu.VMEM((_ROWS_PER_W,), jnp.int32),
        pltpu.VMEM((_ROWS_PER_W, STATE_DIM), jnp.float32),
        pltpu.SemaphoreType.DMA,
    ],
)
def _sc_gather(table_hbm, idx_hbm, out_hbm, idx_v, rows_v, sem):
    wid = lax.axis_index("s") * _NC + lax.axis_index("c")
    base = wid * _ROWS_PER_W
    pltpu.sync_copy(idx_hbm.at[pl.ds(base, _ROWS_PER_W)], idx_v)
    pltpu.async_copy(table_hbm.at[idx_v], rows_v, sem).wait()
    pltpu.sync_copy(rows_v, out_hbm.at[pl.ds(base, _ROWS_PER_W)])


def _patch_body(mean_ref, g_ref, o_ref):
    rows = lax.broadcasted_iota(jnp.int32, (8, STATE_DIM), 0)
    o_ref[...] = jnp.where(rows == 0, mean_ref[...], g_ref[...])


_patch_call = pl.pallas_call(
    _patch_body,
    grid=(1,),
    in_specs=[
        pl.BlockSpec((1, STATE_DIM), lambda i: (0, 0)),
        pl.BlockSpec((8, STATE_DIM), lambda i: (0, 0)),
    ],
    out_specs=pl.BlockSpec((8, STATE_DIM), lambda i: (0, 0)),
    out_shape=jax.ShapeDtypeStruct((N_OUT, STATE_DIM), jnp.float32),
    input_output_aliases={1: 0},
)


def kernel(state, state_buffer, n, ptr):
    n = jnp.asarray(n, jnp.int32)
    ptr = jnp.asarray(ptr, jnp.int32)
    idx = (ptr - (n - N_OUT) - jnp.arange(N_OUT, dtype=jnp.int32)) % BUFFER_SIZE
    gathered = _sc_gather(state_buffer, idx)
    mean2d = _mean_call(state)
    return _patch_call(mean2d, gathered)


# speedup vs baseline: 1.0711x; 1.0028x over previous
import functools

import jax
import jax.numpy as jnp
from jax import lax
from jax.experimental import compute_on
from jax.experimental import pallas as pl
from jax.experimental.pallas import tpu as pltpu
from jax.experimental.pallas import tpu_sc as plsc

STATE_DIM = 1024
BUFFER_SIZE = 8192
N_OUT = 2048
STATE_ROWS = 16384

_CHUNK = 4096
_GRID = STATE_ROWS // _CHUNK

_NC = 2
_NS = 16
_NW = _NC * _NS
_ROWS_PER_W = N_OUT // _NW


def _mean_body(x_ref, o_ref):
    i = pl.program_id(0)
    part = jnp.sum(x_ref[...], axis=0, keepdims=True)

    @pl.when(i == 0)
    def _():
        o_ref[...] = part

    @pl.when(i > 0)
    def _():
        o_ref[...] += part

    @pl.when(i == _GRID - 1)
    def _():
        o_ref[...] *= jnp.float32(1.0 / STATE_ROWS)


_mean_call = pl.pallas_call(
    _mean_body,
    grid=(_GRID,),
    in_specs=[pl.BlockSpec((_CHUNK, STATE_DIM), lambda i: (i, 0))],
    out_specs=pl.BlockSpec((1, STATE_DIM), lambda i: (0, 0)),
    out_shape=jax.ShapeDtypeStruct((1, STATE_DIM), jnp.float32),
)


_sc_mesh = plsc.VectorSubcoreMesh(
    core_axis_name="c", subcore_axis_name="s", num_cores=_NC, num_subcores=_NS
)


@functools.partial(
    pl.kernel,
    out_type=jax.ShapeDtypeStruct((N_OUT, STATE_DIM), jnp.float32),
    mesh=_sc_mesh,
    scratch_types=[
        pltpu.VMEM((_ROWS_PER_W,), jnp.int32),
        pltpu.VMEM((_ROWS_PER_W, STATE_DIM), jnp.float32),
        pltpu.SemaphoreType.DMA,
    ],
)
def _sc_gather(table_hbm, idx_hbm, out_hbm, idx_v, rows_v, sem):
    wid = lax.axis_index("s") * _NC + lax.axis_index("c")
    base = wid * _ROWS_PER_W
    pltpu.sync_copy(idx_hbm.at[pl.ds(base, _ROWS_PER_W)], idx_v)
    pltpu.async_copy(table_hbm.at[idx_v], rows_v, sem).wait()
    pltpu.sync_copy(rows_v, out_hbm.at[pl.ds(base, _ROWS_PER_W)])


def _patch_body(mean_ref, g_ref, o_ref):
    rows = lax.broadcasted_iota(jnp.int32, (8, STATE_DIM), 0)
    o_ref[...] = jnp.where(rows == 0, mean_ref[...], g_ref[...])


_patch_call = pl.pallas_call(
    _patch_body,
    grid=(1,),
    in_specs=[
        pl.BlockSpec((1, STATE_DIM), lambda i: (0, 0)),
        pl.BlockSpec((8, STATE_DIM), lambda i: (0, 0)),
    ],
    out_specs=pl.BlockSpec((8, STATE_DIM), lambda i: (0, 0)),
    out_shape=jax.ShapeDtypeStruct((N_OUT, STATE_DIM), jnp.float32),
    input_output_aliases={1: 0},
)


def kernel(state, state_buffer, n, ptr):
    n = jnp.asarray(n, jnp.int32)
    ptr = jnp.asarray(ptr, jnp.int32)
    idx = (ptr - (n - N_OUT) - jnp.arange(N_OUT, dtype=jnp.int32)) % BUFFER_SIZE
    with compute_on.compute_on("tpu_sparsecore"):
        gathered = _sc_gather(state_buffer, idx)
    mean2d = _mean_call(state)
    return _patch_call(mean2d, gathered)


# speedup vs baseline: 1.1026x; 1.0294x over previous
import functools

import jax
import jax.numpy as jnp
from jax import lax
from jax.experimental import compute_on
from jax.experimental import pallas as pl
from jax.experimental.pallas import tpu as pltpu
from jax.experimental.pallas import tpu_sc as plsc

STATE_DIM = 1024
BUFFER_SIZE = 8192
N_OUT = 2048
STATE_ROWS = 16384

_CHUNK = 4096
_GRID = STATE_ROWS // _CHUNK

_NC = 1
_NS = 16
_NW = _NC * _NS
_ROWS_PER_W = N_OUT // _NW
_GPASS_ROWS = 64
_NGPASS = _ROWS_PER_W // _GPASS_ROWS


def _mean_body(x_ref, o_ref):
    i = pl.program_id(0)
    part = jnp.sum(x_ref[...], axis=0, keepdims=True)

    @pl.when(i == 0)
    def _():
        o_ref[...] = part

    @pl.when(i > 0)
    def _():
        o_ref[...] += part

    @pl.when(i == _GRID - 1)
    def _():
        o_ref[...] *= jnp.float32(1.0 / STATE_ROWS)


_mean_call = pl.pallas_call(
    _mean_body,
    grid=(_GRID,),
    in_specs=[pl.BlockSpec((_CHUNK, STATE_DIM), lambda i: (i, 0))],
    out_specs=pl.BlockSpec((1, STATE_DIM), lambda i: (0, 0)),
    out_shape=jax.ShapeDtypeStruct((1, STATE_DIM), jnp.float32),
)


_sc_mesh = plsc.VectorSubcoreMesh(
    core_axis_name="c", subcore_axis_name="s", num_cores=_NC, num_subcores=_NS
)


@functools.partial(
    pl.kernel,
    out_type=jax.ShapeDtypeStruct((N_OUT, STATE_DIM), jnp.float32),
    mesh=_sc_mesh,
    scratch_types=[
        pltpu.VMEM((_GPASS_ROWS,), jnp.int32),
        pltpu.VMEM((_GPASS_ROWS, STATE_DIM), jnp.float32),
        pltpu.SemaphoreType.DMA,
    ],
)
def _sc_gather(table_hbm, idx_hbm, out_hbm, idx_v, rows_v, sem):
    wid = lax.axis_index("s") * _NC + lax.axis_index("c")
    for p in range(_NGPASS):
        base = wid * _ROWS_PER_W + p * _GPASS_ROWS
        pltpu.sync_copy(idx_hbm.at[pl.ds(base, _GPASS_ROWS)], idx_v)
        pltpu.async_copy(table_hbm.at[idx_v], rows_v, sem).wait()
        pltpu.sync_copy(rows_v, out_hbm.at[pl.ds(base, _GPASS_ROWS)])


def _patch_body(mean_ref, g_ref, o_ref):
    rows = lax.broadcasted_iota(jnp.int32, (8, STATE_DIM), 0)
    o_ref[...] = jnp.where(rows == 0, mean_ref[...], g_ref[...])


_patch_call = pl.pallas_call(
    _patch_body,
    grid=(1,),
    in_specs=[
        pl.BlockSpec((1, STATE_DIM), lambda i: (0, 0)),
        pl.BlockSpec((8, STATE_DIM), lambda i: (0, 0)),
    ],
    out_specs=pl.BlockSpec((8, STATE_DIM), lambda i: (0, 0)),
    out_shape=jax.ShapeDtypeStruct((N_OUT, STATE_DIM), jnp.float32),
    input_output_aliases={1: 0},
)


def kernel(state, state_buffer, n, ptr):
    n = jnp.asarray(n, jnp.int32)
    ptr = jnp.asarray(ptr, jnp.int32)
    idx = (ptr - (n - N_OUT) - jnp.arange(N_OUT, dtype=jnp.int32)) % BUFFER_SIZE
    with compute_on.compute_on("tpu_sparsecore"):
        gathered = _sc_gather(state_buffer, idx)
    mean2d = _mean_call(state)
    return _patch_call(mean2d, gathered)


# speedup vs baseline: 1.1037x; 1.0010x over previous
import functools

import jax
import jax.numpy as jnp
from jax import lax
from jax.experimental import compute_on
from jax.experimental import pallas as pl
from jax.experimental.pallas import tpu as pltpu
from jax.experimental.pallas import tpu_sc as plsc

STATE_DIM = 1024
BUFFER_SIZE = 8192
N_OUT = 2048
STATE_ROWS = 16384

_CHUNK = 4096
_GRID = STATE_ROWS // _CHUNK

_NC = 1
_NS = 16
_NW = _NC * _NS
_ROWS_PER_W = N_OUT // _NW
_GPASS_ROWS = 64
_NGPASS = _ROWS_PER_W // _GPASS_ROWS


def _mean_body(x_ref, o_ref):
    i = pl.program_id(0)
    part = jnp.sum(x_ref[...], axis=0, keepdims=True)

    @pl.when(i == 0)
    def _():
        o_ref[...] = part

    @pl.when(i > 0)
    def _():
        o_ref[...] += part

    @pl.when(i == _GRID - 1)
    def _():
        o_ref[...] *= jnp.float32(1.0 / STATE_ROWS)


_mean_call = pl.pallas_call(
    _mean_body,
    grid=(_GRID,),
    in_specs=[pl.BlockSpec((_CHUNK, STATE_DIM), lambda i: (i, 0))],
    out_specs=pl.BlockSpec((1, STATE_DIM), lambda i: (0, 0)),
    out_shape=jax.ShapeDtypeStruct((1, STATE_DIM), jnp.float32),
)


_sc_mesh = plsc.VectorSubcoreMesh(
    core_axis_name="c", subcore_axis_name="s", num_cores=_NC, num_subcores=_NS
)


@functools.partial(
    pl.kernel,
    out_type=jax.ShapeDtypeStruct((N_OUT, STATE_DIM), jnp.float32),
    mesh=_sc_mesh,
    cost_estimate=pl.CostEstimate(
        flops=0,
        transcendentals=0,
        bytes_accessed=2 * N_OUT * STATE_DIM * 4,
    ),
    scratch_types=[
        pltpu.VMEM((_GPASS_ROWS,), jnp.int32),
        pltpu.VMEM((_GPASS_ROWS, STATE_DIM), jnp.float32),
        pltpu.SemaphoreType.DMA,
    ],
)
def _sc_gather(table_hbm, idx_hbm, out_hbm, idx_v, rows_v, sem):
    wid = lax.axis_index("s") * _NC + lax.axis_index("c")
    for p in range(_NGPASS):
        base = wid * _ROWS_PER_W + p * _GPASS_ROWS
        pltpu.sync_copy(idx_hbm.at[pl.ds(base, _GPASS_ROWS)], idx_v)
        pltpu.async_copy(table_hbm.at[idx_v], rows_v, sem).wait()
        pltpu.sync_copy(rows_v, out_hbm.at[pl.ds(base, _GPASS_ROWS)])


def _patch_body(mean_ref, g_ref, o_ref):
    rows = lax.broadcasted_iota(jnp.int32, (8, STATE_DIM), 0)
    o_ref[...] = jnp.where(rows == 0, mean_ref[...], g_ref[...])


_patch_call = pl.pallas_call(
    _patch_body,
    grid=(1,),
    in_specs=[
        pl.BlockSpec((1, STATE_DIM), lambda i: (0, 0)),
        pl.BlockSpec((8, STATE_DIM), lambda i: (0, 0)),
    ],
    out_specs=pl.BlockSpec((8, STATE_DIM), lambda i: (0, 0)),
    out_shape=jax.ShapeDtypeStruct((N_OUT, STATE_DIM), jnp.float32),
    input_output_aliases={1: 0},
)


def kernel(state, state_buffer, n, ptr):
    n = jnp.asarray(n, jnp.int32)
    ptr = jnp.asarray(ptr, jnp.int32)
    idx = (ptr - (n - N_OUT) - jnp.arange(N_OUT, dtype=jnp.int32)) % BUFFER_SIZE
    with compute_on.compute_on("tpu_sparsecore"):
        gathered = _sc_gather(state_buffer, idx)
    mean2d = _mean_call(state)
    return _patch_call(mean2d, gathered)
